# baseline (speedup 1.0000x reference)
import functools

import jax
import jax.numpy as jnp
from jax import lax
from jax.experimental import pallas as pl
from jax.experimental.pallas import tpu as pltpu
from jax.experimental.pallas import tpu_sc as plsc

_B, _N, _DIN, _E = 16, 1024, 2, 128
_K = 10
_R = 1024
_L = 16
_NW = 32
_HALVES = 2
_BH = _B // _HALVES
_TOTH = _BH * _N * _K
_BPW = _TOTH // _NW
_CH = 128
_NCH = _BPW // _CH


def _make_select_body(boffset):
    def _select_body(xr_ref, xt_ref, out_ref):
        b = pl.program_id(0)
        xr = xr_ref[0]
        xt = xt_ref[0]

        x0c = xt[0:1, :]
        x1c = xt[1:2, :]
        sq_c = x0c * x0c + x1c * x1c
        x0r = xr[:, 0:1]
        x1r = xr[:, 1:2]
        sq_r = x0r * x0r + x1r * x1r
        dot = jax.lax.dot(xr, xt, preferred_element_type=jnp.float32)
        d2 = (sq_r + sq_c) - 2.0 * dot
        vals = jnp.maximum(d2, 1e-12)

        iotaf = (jax.lax.broadcasted_iota(jnp.int32, (_R, _N), 1)
                 .astype(jnp.float32))
        sels = []
        for _ in range(_K):
            vmin = jnp.min(vals, axis=1, keepdims=True)
            tie = vals == vmin
            sel = jnp.max(jnp.where(tie, iotaf, -1.0), axis=1, keepdims=True)
            onehot = iotaf == sel
            vals = jnp.where(onehot, jnp.float32(jnp.inf), vals)
            sels.append(sel)
        selcat = jnp.concatenate(sels, axis=1)
        out_ref[0] = selcat.astype(jnp.int32) + (b + boffset) * _N

    return _select_body


def _sc_gather(table, idx):
    mesh = plsc.VectorSubcoreMesh(core_axis_name="c", subcore_axis_name="s")

    @functools.partial(
        pl.kernel, mesh=mesh,
        compiler_params=pltpu.CompilerParams(use_tc_tiling_on_sc=False),
        out_type=jax.ShapeDtypeStruct((_TOTH, _L), jnp.float32),
        scratch_types=[
            pltpu.VMEM((_BPW,), jnp.int32),
            pltpu.VMEM((_BPW, _L), jnp.float32),
            pltpu.SemaphoreType.DMA,
        ],
    )
    def k(table_hbm, idx_hbm, out_hbm, idx_v, rows_v, sem):
        wid = lax.axis_index("s") * 2 + lax.axis_index("c")
        base = wid * _BPW
        pltpu.sync_copy(idx_hbm.at[pl.ds(base, _BPW)], idx_v)

        def fire(i, _):
            off = i * _CH
            pltpu.async_copy(
                table_hbm.at[idx_v.at[pl.ds(off, _CH)]],
                rows_v.at[pl.ds(off, _CH)], sem)
            return 0

        def drain(i, _):
            off = i * _CH
            pltpu.make_async_copy(
                table_hbm.at[idx_v.at[pl.ds(off, _CH)]],
                rows_v.at[pl.ds(off, _CH)], sem).wait()
            return 0

        lax.fori_loop(0, _NCH, fire, 0)
        lax.fori_loop(0, _NCH, drain, 0)
        pltpu.sync_copy(rows_v, out_hbm.at[pl.ds(base, _BPW)])

    return k(table, idx)


def _dense_body(xr_ref, win_ref, w1_ref, wcvp_ref, w2_ref, b1p_ref, bcv_ref,
                out_ref):
    wfold = jax.lax.dot(wcvp_ref[...], w2_ref[...])
    bias = b1p_ref[...] + jax.lax.dot(bcv_ref[...], w2_ref[...])
    out_ref[...] = (jax.lax.dot(xr_ref[...], w1_ref[...])
                    + jax.lax.dot(win_ref[...], wfold) + bias)


def kernel(x, W1, b1, Wconv, bconv, W2, b2):
    xt = jnp.transpose(x, (0, 2, 1))
    table = jnp.pad(x.reshape(_B * _N, _DIN), ((0, 0), (0, _L - _DIN)))

    wins = []
    for h in range(_HALVES):
        idx_h = pl.pallas_call(
            _make_select_body(h * _BH),
            grid=(_BH, _N // _R),
            in_specs=[
                pl.BlockSpec((1, _R, _DIN), lambda b, r: (b, r, 0)),
                pl.BlockSpec((1, _DIN, _N), lambda b, r: (b, 0, 0)),
            ],
            out_specs=pl.BlockSpec((1, _R, _K), lambda b, r: (b, r, 0)),
            out_shape=jax.ShapeDtypeStruct((_BH, _N, _K), jnp.int32),
        )(x[h * _BH:(h + 1) * _BH], xt[h * _BH:(h + 1) * _BH])
        wins.append(_sc_gather(table, idx_h.reshape(_TOTH)))
    win = jnp.concatenate(wins, axis=0).reshape(_B * _N, _K * _L)

    wcvp = jnp.pad(jnp.transpose(Wconv, (2, 1, 0))[::-1],
                   ((0, 0), (0, _L - _DIN), (0, 0))).reshape(_K * _L, _E)
    b1p = (b1 + b2).reshape(1, _E)
    bcv = bconv.reshape(1, _E)

    _RT = 2048
    out = pl.pallas_call(
        _dense_body,
        grid=(_B * _N // _RT,),
        in_specs=[
            pl.BlockSpec((_RT, _DIN), lambda r: (r, 0)),
            pl.BlockSpec((_RT, _K * _L), lambda r: (r, 0)),
            pl.BlockSpec((_DIN, _E), lambda r: (0, 0)),
            pl.BlockSpec((_K * _L, _E), lambda r: (0, 0)),
            pl.BlockSpec((_E, _E), lambda r: (0, 0)),
            pl.BlockSpec((1, _E), lambda r: (0, 0)),
            pl.BlockSpec((1, _E), lambda r: (0, 0)),
        ],
        out_specs=pl.BlockSpec((_RT, _E), lambda r: (r, 0)),
        out_shape=jax.ShapeDtypeStruct((_B * _N, _E), jnp.float32),
    )(x.reshape(_B * _N, _DIN), win, W1, wcvp, W2, b1p, bcv)
    return out.reshape(_B, _N, _E)


# speedup vs baseline: 1.6451x; 1.6451x over previous
import functools

import jax
import jax.numpy as jnp
from jax import lax
from jax.experimental import pallas as pl
from jax.experimental.pallas import tpu as pltpu
from jax.experimental.pallas import tpu_sc as plsc

_B, _N, _DIN, _E = 16, 1024, 2, 128
_K = 10
_R = 1024
_L = 16
_NW = 32
_TOT = _B * _N * _K
_BPW = _TOT // _NW
_CH = 128
_NCH = _BPW // _CH


def _select_body(xr_ref, xt_ref, out_ref):
    b = pl.program_id(0)
    xr = xr_ref[0]
    xt = xt_ref[0]

    x0c = xt[0:1, :]
    x1c = xt[1:2, :]
    sq_c = x0c * x0c + x1c * x1c
    x0r = xr[:, 0:1]
    x1r = xr[:, 1:2]
    sq_r = x0r * x0r + x1r * x1r
    dot = jax.lax.dot(xr, xt, preferred_element_type=jnp.float32)
    d2 = (sq_r + sq_c) - 2.0 * dot
    vals = jnp.maximum(d2, 1e-12)

    iotaf = jax.lax.broadcasted_iota(jnp.int32, (_R, _N), 1).astype(jnp.float32)
    sels = []
    for m in range(_K):
        vmin = jnp.min(vals, axis=1, keepdims=True)
        tie = vals == vmin
        sel = jnp.max(jnp.where(tie, iotaf, -1.0), axis=1, keepdims=True)
        sels.append(sel)
        if m < _K - 1:
            onehot = iotaf == sel
            vals = jnp.where(onehot, jnp.float32(jnp.inf), vals)
    selcat = jnp.concatenate(sels, axis=1)
    out_ref[0] = selcat.astype(jnp.int32) + b * _N


def _sc_gather(table, idx):
    mesh = plsc.VectorSubcoreMesh(core_axis_name="c", subcore_axis_name="s")

    @functools.partial(
        pl.kernel, mesh=mesh,
        compiler_params=pltpu.CompilerParams(use_tc_tiling_on_sc=False),
        out_type=jax.ShapeDtypeStruct((_TOT, _L), jnp.float32),
        scratch_types=[
            pltpu.VMEM((_BPW,), jnp.int32),
            pltpu.VMEM((_BPW, _L), jnp.float32),
            pltpu.SemaphoreType.DMA,
        ],
    )
    def k(table_hbm, idx_hbm, out_hbm, idx_v, rows_v, sem):
        wid = lax.axis_index("s") * 2 + lax.axis_index("c")
        base = wid * _BPW
        pltpu.sync_copy(idx_hbm.at[pl.ds(base, _BPW)], idx_v)

        def fire(i, _):
            off = i * _CH
            pltpu.async_copy(
                table_hbm.at[idx_v.at[pl.ds(off, _CH)]],
                rows_v.at[pl.ds(off, _CH)], sem)
            return 0

        def drain(i, _):
            off = i * _CH
            pltpu.make_async_copy(
                table_hbm.at[idx_v.at[pl.ds(off, _CH)]],
                rows_v.at[pl.ds(off, _CH)], sem).wait()
            return 0

        lax.fori_loop(0, _NCH, fire, 0)
        lax.fori_loop(0, _NCH, drain, 0)
        pltpu.sync_copy(rows_v, out_hbm.at[pl.ds(base, _BPW)])

    return k(table, idx)


def _dense_body(xr_ref, win_ref, w1_ref, wcvp_ref, w2_ref, b1p_ref, bcv_ref,
                out_ref):
    wfold = jax.lax.dot(wcvp_ref[...], w2_ref[...])
    bias = b1p_ref[...] + jax.lax.dot(bcv_ref[...], w2_ref[...])
    out_ref[...] = (jax.lax.dot(xr_ref[...], w1_ref[...])
                    + jax.lax.dot(win_ref[...], wfold) + bias)


def kernel(x, W1, b1, Wconv, bconv, W2, b2):
    xt = jnp.transpose(x, (0, 2, 1))

    idx = pl.pallas_call(
        _select_body,
        grid=(_B, _N // _R),
        in_specs=[
            pl.BlockSpec((1, _R, _DIN), lambda b, r: (b, r, 0)),
            pl.BlockSpec((1, _DIN, _N), lambda b, r: (b, 0, 0)),
        ],
        out_specs=pl.BlockSpec((1, _R, _K), lambda b, r: (b, r, 0)),
        out_shape=jax.ShapeDtypeStruct((_B, _N, _K), jnp.int32),
    )(x, xt)

    table = jnp.pad(x.reshape(_B * _N, _DIN), ((0, 0), (0, _L - _DIN)))
    win = _sc_gather(table, idx.reshape(_TOT))
    win = win.reshape(_B * _N, _K * _L)

    wcvp = jnp.pad(jnp.transpose(Wconv, (2, 1, 0))[::-1],
                   ((0, 0), (0, _L - _DIN), (0, 0))).reshape(_K * _L, _E)
    b1p = (b1 + b2).reshape(1, _E)
    bcv = bconv.reshape(1, _E)

    _RT = 2048
    out = pl.pallas_call(
        _dense_body,
        grid=(_B * _N // _RT,),
        in_specs=[
            pl.BlockSpec((_RT, _DIN), lambda r: (r, 0)),
            pl.BlockSpec((_RT, _K * _L), lambda r: (r, 0)),
            pl.BlockSpec((_DIN, _E), lambda r: (0, 0)),
            pl.BlockSpec((_K * _L, _E), lambda r: (0, 0)),
            pl.BlockSpec((_E, _E), lambda r: (0, 0)),
            pl.BlockSpec((1, _E), lambda r: (0, 0)),
            pl.BlockSpec((1, _E), lambda r: (0, 0)),
        ],
        out_specs=pl.BlockSpec((_RT, _E), lambda r: (r, 0)),
        out_shape=jax.ShapeDtypeStruct((_B * _N, _E), jnp.float32),
    )(x.reshape(_B * _N, _DIN), win, W1, wcvp, W2, b1p, bcv)
    return out.reshape(_B, _N, _E)
